# baseline (speedup 1.0000x reference)
import jax
import jax.numpy as jnp
from jax.experimental import pallas as pl
from jax.experimental.pallas import tpu as pltpu

B = 16384
IN = 4096
H = 128
K = 1024
D = 32

BM1 = 512
BM2 = 2048
NB1 = B // BM1
NB2 = B // BM2


def _row_sum32(a):
    p = a[:, 0:8]
    for i in range(1, 4):
        p = p + a[:, i * 8:(i + 1) * 8]
    while p.shape[1] > 1:
        h = p.shape[1] // 2
        p = p[:, :h] + p[:, h:]
    return p


def _enc_kernel(s0_ref, s1_ref, w0_ref, b0_ref, w1_ref, b1_ref, x_ref):
    x = (jnp.dot(s0_ref[...], w0_ref[...],
                 preferred_element_type=jnp.float32) + b0_ref[...]) + \
        (jnp.dot(s1_ref[...], w1_ref[...],
                 preferred_element_type=jnp.float32) + b1_ref[...])
    x_ref[...] = jnp.maximum(x, 0.0)


def _vq_kernel(x_ref, wq_ref, bq_ref, emb_ref, qst_ref, loss_ref, idx_ref,
               ppl_ref, cnt_ref):
    i = pl.program_id(0)

    f = jnp.dot(x_ref[...], wq_ref[...],
                preferred_element_type=jnp.float32) + bq_ref[...]
    e = emb_ref[...]
    t1 = _row_sum32(f * f)
    t2 = _row_sum32(e * e).reshape(1, K)
    t3 = jnp.dot(f, e.T, preferred_element_type=jnp.float32)
    dist = (t1 + t2) - 2.0 * t3
    iota = jax.lax.broadcasted_iota(jnp.int32, (BM2, K), 1)
    minv = jnp.min(dist, axis=1, keepdims=True)
    idx = jnp.min(jnp.where(dist == minv, iota, K), axis=1)
    onehot = (iota == idx[:, None]).astype(jnp.float32)
    q = jnp.dot(onehot, e, preferred_element_type=jnp.float32)

    diff = q - f
    m = _row_sum32(diff * diff) * (1.0 / D)
    loss_ref[...] = m + m
    qst_ref[...] = f + diff
    idx_ref[...] = idx[:, None]

    csum = jnp.sum(onehot, axis=0)[None, :]

    @pl.when(i == 0)
    def _init():
        cnt_ref[...] = csum

    @pl.when(i > 0)
    def _acc():
        cnt_ref[...] += csum

    @pl.when(i == NB2 - 1)
    def _fin():
        p = cnt_ref[...] * (1.0 / B)
        ppl_ref[...] = jnp.exp(-jnp.sum(p * jnp.log(p + 1e-10))).reshape(1, 1)


@jax.jit
def kernel(s0, s1, W0, b0, W1, b1, Wq, bq, emb):
    x = pl.pallas_call(
        _enc_kernel,
        grid=(NB1,),
        in_specs=[
            pl.BlockSpec((BM1, IN), lambda i: (i, 0)),
            pl.BlockSpec((BM1, IN), lambda i: (i, 0)),
            pl.BlockSpec((IN, H), lambda i: (0, 0)),
            pl.BlockSpec((1, H), lambda i: (0, 0)),
            pl.BlockSpec((IN, H), lambda i: (0, 0)),
            pl.BlockSpec((1, H), lambda i: (0, 0)),
        ],
        out_specs=pl.BlockSpec((BM1, H), lambda i: (i, 0)),
        out_shape=jax.ShapeDtypeStruct((B, H), jnp.float32),
    )(s0, s1, W0, b0.reshape(1, H), W1, b1.reshape(1, H))

    qst, loss, idx, ppl = pl.pallas_call(
        _vq_kernel,
        grid=(NB2,),
        in_specs=[
            pl.BlockSpec((BM2, H), lambda i: (i, 0)),
            pl.BlockSpec((H, D), lambda i: (0, 0)),
            pl.BlockSpec((1, D), lambda i: (0, 0)),
            pl.BlockSpec((K, D), lambda i: (0, 0)),
        ],
        out_specs=[
            pl.BlockSpec((BM2, D), lambda i: (i, 0)),
            pl.BlockSpec((BM2, 1), lambda i: (i, 0)),
            pl.BlockSpec((BM2, 1), lambda i: (i, 0)),
            pl.BlockSpec((1, 1), lambda i: (0, 0)),
        ],
        out_shape=[
            jax.ShapeDtypeStruct((B, D), jnp.float32),
            jax.ShapeDtypeStruct((B, 1), jnp.float32),
            jax.ShapeDtypeStruct((B, 1), jnp.int32),
            jax.ShapeDtypeStruct((1, 1), jnp.float32),
        ],
        scratch_shapes=[pltpu.VMEM((1, K), jnp.float32)],
    )(x, Wq, bq.reshape(1, D), emb)

    return (qst, loss.reshape(B), ppl[0, 0], idx.reshape(B))


# speedup vs baseline: 1.1557x; 1.1557x over previous
import jax
import jax.numpy as jnp
from jax.experimental import pallas as pl
from jax.experimental.pallas import tpu as pltpu

B = 16384
IN = 4096
H = 128
K = 1024
D = 32

BM = 512
NB = B // BM


def _row_sum32(a):
    p = a[:, 0:8]
    for i in range(1, 4):
        p = p + a[:, i * 8:(i + 1) * 8]
    while p.shape[1] > 1:
        h = p.shape[1] // 2
        p = p[:, :h] + p[:, h:]
    return p


def _vq_kernel(s0_ref, s1_ref, w0_ref, b0_ref, w1_ref, b1_ref, wq_ref,
               bq_ref, emb_ref, qst_ref, loss_ref, idx_ref, ppl_ref,
               xbuf_ref, cnt_ref):
    i = pl.program_id(0)

    @pl.when(i < NB)
    def _stage1():
        x = (jnp.dot(s0_ref[...], w0_ref[...],
                     preferred_element_type=jnp.float32) + b0_ref[...]) + \
            (jnp.dot(s1_ref[...], w1_ref[...],
                     preferred_element_type=jnp.float32) + b1_ref[...])
        xbuf_ref[i % 2] = jnp.maximum(x, 0.0)

    @pl.when(i > 0)
    def _stage2():
        f = jnp.dot(xbuf_ref[(i - 1) % 2], wq_ref[...],
                    preferred_element_type=jnp.float32) + bq_ref[...]
        e = emb_ref[...]
        t1 = _row_sum32(f * f)
        t2 = _row_sum32(e * e).reshape(1, K)
        t3 = jnp.dot(f, e.T, preferred_element_type=jnp.float32)
        dist = (t1 + t2) - 2.0 * t3
        iota = jax.lax.broadcasted_iota(jnp.int32, (BM, K), 1)
        minv = jnp.min(dist, axis=1, keepdims=True)
        idx = jnp.min(jnp.where(dist == minv, iota, K), axis=1)
        onehot = (iota == idx[:, None]).astype(jnp.float32)
        q = jnp.dot(onehot, e, preferred_element_type=jnp.float32)
        diff = q - f
        m = _row_sum32(diff * diff) * (1.0 / D)
        loss_ref[...] = m + m
        qst_ref[...] = f + diff
        idx_ref[...] = idx[:, None]
        csum = jnp.sum(onehot, axis=0)[None, :]

        @pl.when(i == 1)
        def _init():
            cnt_ref[...] = csum

        @pl.when(i > 1)
        def _acc():
            cnt_ref[...] += csum

        @pl.when(i == NB)
        def _fin():
            p = cnt_ref[...] * (1.0 / B)
            ppl_ref[...] = jnp.exp(-jnp.sum(p * jnp.log(p + 1e-10))).reshape(1, 1)


@jax.jit
def kernel(s0, s1, W0, b0, W1, b1, Wq, bq, emb):
    in_i = lambda i: (jnp.minimum(i, NB - 1), 0)
    out_j = lambda i: (jnp.maximum(i - 1, 0), 0)
    qst, loss, idx, ppl = pl.pallas_call(
        _vq_kernel,
        grid=(NB + 1,),
        in_specs=[
            pl.BlockSpec((BM, IN), in_i),
            pl.BlockSpec((BM, IN), in_i),
            pl.BlockSpec((IN, H), lambda i: (0, 0)),
            pl.BlockSpec((1, H), lambda i: (0, 0)),
            pl.BlockSpec((IN, H), lambda i: (0, 0)),
            pl.BlockSpec((1, H), lambda i: (0, 0)),
            pl.BlockSpec((H, D), lambda i: (0, 0)),
            pl.BlockSpec((1, D), lambda i: (0, 0)),
            pl.BlockSpec((K, D), lambda i: (0, 0)),
        ],
        out_specs=[
            pl.BlockSpec((BM, D), out_j),
            pl.BlockSpec((BM, 1), out_j),
            pl.BlockSpec((BM, 1), out_j),
            pl.BlockSpec((1, 1), lambda i: (0, 0)),
        ],
        out_shape=[
            jax.ShapeDtypeStruct((B, D), jnp.float32),
            jax.ShapeDtypeStruct((B, 1), jnp.float32),
            jax.ShapeDtypeStruct((B, 1), jnp.int32),
            jax.ShapeDtypeStruct((1, 1), jnp.float32),
        ],
        scratch_shapes=[pltpu.VMEM((2, BM, H), jnp.float32),
                        pltpu.VMEM((1, K), jnp.float32)],
    )(s0, s1, W0, b0.reshape(1, H), W1, b1.reshape(1, H), Wq,
      bq.reshape(1, D), emb)
    return (qst, loss.reshape(B), ppl[0, 0], idx.reshape(B))


# speedup vs baseline: 1.1703x; 1.0126x over previous
import jax
import jax.numpy as jnp
from jax.experimental import pallas as pl
from jax.experimental.pallas import tpu as pltpu

B = 16384
IN = 4096
H = 128
K = 1024
D = 32

BM = 512
NB = B // BM


def _row_sum32(a):
    p = a[:, 0:8]
    for i in range(1, 4):
        p = p + a[:, i * 8:(i + 1) * 8]
    while p.shape[1] > 1:
        h = p.shape[1] // 2
        p = p[:, :h] + p[:, h:]
    return p


def _vq_kernel(s0_ref, s1_ref, w0_ref, b0_ref, w1_ref, b1_ref, wq_ref,
               bq_ref, emb_ref, qst_ref, loss_ref, idx_ref, ppl_ref,
               xbuf_ref, cnt_ref, t2s_ref):
    i = pl.program_id(0)

    @pl.when(i < NB)
    def _stage1():
        x = (jnp.dot(s0_ref[...], w0_ref[...],
                     preferred_element_type=jnp.float32) + b0_ref[...]) + \
            (jnp.dot(s1_ref[...], w1_ref[...],
                     preferred_element_type=jnp.float32) + b1_ref[...])
        xbuf_ref[i % 2] = jnp.maximum(x, 0.0)

    @pl.when(i == 0)
    def _prep():
        e = emb_ref[...]
        t2s_ref[...] = _row_sum32(e * e).reshape(1, K)

    @pl.when(i > 0)
    def _stage2():
        f = jnp.dot(xbuf_ref[(i - 1) % 2], wq_ref[...],
                    preferred_element_type=jnp.float32) + bq_ref[...]
        e = emb_ref[...]
        t1 = _row_sum32(f * f)
        t2 = t2s_ref[...]
        t3d = jnp.dot(f, (e + e).T, preferred_element_type=jnp.float32)
        dist = (t1 + t2) - t3d
        iota = jax.lax.broadcasted_iota(jnp.int32, (BM, K), 1)
        minv = jnp.min(dist, axis=1, keepdims=True)
        idx = jnp.min(jnp.where(dist == minv, iota, K), axis=1)
        onehot = (iota == idx[:, None]).astype(jnp.float32)
        q = jnp.dot(onehot, e, preferred_element_type=jnp.float32)
        diff = q - f
        m = _row_sum32(diff * diff) * (1.0 / D)
        loss_ref[...] = m + m
        qst_ref[...] = f + diff
        idx_ref[...] = idx[:, None]
        ones = jnp.ones((1, BM), jnp.float32)
        csum = jnp.dot(ones, onehot, preferred_element_type=jnp.float32)

        @pl.when(i == 1)
        def _init():
            cnt_ref[...] = csum

        @pl.when(i > 1)
        def _acc():
            cnt_ref[...] += csum

        @pl.when(i == NB)
        def _fin():
            p = cnt_ref[...] * (1.0 / B)
            ppl_ref[...] = jnp.exp(-jnp.sum(p * jnp.log(p + 1e-10))).reshape(1, 1)


@jax.jit
def kernel(s0, s1, W0, b0, W1, b1, Wq, bq, emb):
    in_i = lambda i: (jnp.minimum(i, NB - 1), 0)
    out_j = lambda i: (jnp.maximum(i - 1, 0), 0)
    qst, loss, idx, ppl = pl.pallas_call(
        _vq_kernel,
        grid=(NB + 1,),
        in_specs=[
            pl.BlockSpec((BM, IN), in_i),
            pl.BlockSpec((BM, IN), in_i),
            pl.BlockSpec((IN, H), lambda i: (0, 0)),
            pl.BlockSpec((1, H), lambda i: (0, 0)),
            pl.BlockSpec((IN, H), lambda i: (0, 0)),
            pl.BlockSpec((1, H), lambda i: (0, 0)),
            pl.BlockSpec((H, D), lambda i: (0, 0)),
            pl.BlockSpec((1, D), lambda i: (0, 0)),
            pl.BlockSpec((K, D), lambda i: (0, 0)),
        ],
        out_specs=[
            pl.BlockSpec((BM, D), out_j),
            pl.BlockSpec((BM, 1), out_j),
            pl.BlockSpec((BM, 1), out_j),
            pl.BlockSpec((1, 1), lambda i: (0, 0)),
        ],
        out_shape=[
            jax.ShapeDtypeStruct((B, D), jnp.float32),
            jax.ShapeDtypeStruct((B, 1), jnp.float32),
            jax.ShapeDtypeStruct((B, 1), jnp.int32),
            jax.ShapeDtypeStruct((1, 1), jnp.float32),
        ],
        scratch_shapes=[pltpu.VMEM((2, BM, H), jnp.float32),
                        pltpu.VMEM((1, K), jnp.float32),
                        pltpu.VMEM((1, K), jnp.float32)],
    )(s0, s1, W0, b0.reshape(1, H), W1, b1.reshape(1, H), Wq,
      bq.reshape(1, D), emb)
    return (qst, loss.reshape(B), ppl[0, 0], idx.reshape(B))
